# baseline (speedup 1.0000x reference)
import functools

import jax
import jax.numpy as jnp
from jax import lax
from jax.experimental import pallas as pl
from jax.experimental.pallas import tpu as pltpu
from jax.experimental.pallas import tpu_sc as plsc

_D = 64
_L = 16
_HIST = 200
_RPT = 16
_NHT = 13
_NIDX = 1024

_mesh = plsc.VectorSubcoreMesh(core_axis_name="c", subcore_axis_name="s")


@functools.partial(
    pl.kernel,
    out_type=jax.ShapeDtypeStruct((24, _D), jnp.float32),
    mesh=_mesh,
    scratch_types=[
        pltpu.VMEM((_L,), jnp.int32),
        pltpu.VMEM((_RPT + 1, 2 * _D), jnp.float32),
        pltpu.VMEM((_RPT, 2 * _D), jnp.float32),
        pltpu.VMEM((2 * _L,), jnp.int32),
        pltpu.VMEM((2 * _L,), jnp.int32),
        pltpu.VMEM((_D,), jnp.float32),
        pltpu.SemaphoreType.DMA,
    ],
    compiler_params=pltpu.CompilerParams(needs_layout_passes=False),
)
def _sc_stage_a(pidx_hbm, utab_hbm, itab_hbm, out_hbm,
                idx_v, rows_v, rows2_v, vbuf, rbuf, part_v, sem):
    cid = lax.axis_index("c")
    sid = lax.axis_index("s")
    on0 = cid == 0
    lane = lax.iota(jnp.int32, _L)

    def gather2(tab):
        cp1 = pltpu.async_copy(tab.at[idx_v], rows_v.at[pl.ds(0, _RPT)], sem)
        cp2 = pltpu.async_copy(tab.at[idx_v], rows2_v, sem)
        splats = [plsc.load_gather(rbuf, [jnp.full((_L,), _L + i, jnp.int32)])
                  for i in range(_RPT)]
        cp1.wait()
        cp2.wait()
        return splats

    @pl.when(jnp.logical_and(on0, sid < _NHT))
    def _():
        base = sid * _RPT
        pltpu.sync_copy(pidx_hbm.at[pl.ds(base, _RPT)], idx_v)
        pltpu.sync_copy(pidx_hbm.at[pl.ds(256 + base, _L)],
                        rbuf.at[pl.ds(_L, _L)])
        pltpu.sync_copy(pidx_hbm.at[pl.ds(768 + base, _L)],
                        vbuf.at[pl.ds(_L, _L)])
        for j in range(2 * _D // _L):
            rows_v[_RPT, pl.ds(_L * j, _L)] = jnp.zeros((_L,), jnp.float32)
        splats = gather2(itab_hbm)
        acc = [jnp.zeros((_L,), jnp.float32) for _ in range(_D // _L)]
        for i in range(_RPT):
            pad = plsc.load_gather(vbuf, [jnp.full((_L,), _L + i, jnp.int32)])
            rowv = jnp.where(pad == 0, jnp.full((_L,), i, jnp.int32),
                             jnp.full((_L,), _RPT, jnp.int32))
            for j in range(_D // _L):
                v = plsc.load_gather(
                    rows_v, [rowv, splats[i] * _D + (lane + _L * j)])
                acc[j] = acc[j] + v
        for j in range(_D // _L):
            part_v[pl.ds(_L * j, _L)] = acc[j]
        pltpu.sync_copy(part_v, out_hbm.at[sid])

    @pl.when(jnp.logical_and(on0, sid >= _NHT))
    def _():
        @pl.when(sid == _NHT)
        def _():
            pltpu.sync_copy(pidx_hbm.at[pl.ds(512, _RPT)], idx_v)

        @pl.when(sid > _NHT)
        def _():
            pltpu.sync_copy(pidx_hbm.at[pl.ds(192, _RPT)], idx_v)

        pltpu.sync_copy(pidx_hbm.at[pl.ds(256 + 192, _L)],
                        rbuf.at[pl.ds(_L, _L)])

        @pl.when(sid == _NHT)
        def _():
            splats = gather2(utab_hbm)
            for j in range(_D // _L):
                v = plsc.load_gather(
                    rows_v, [jnp.zeros((_L,), jnp.int32),
                             splats[10] * _D + (lane + _L * j)])
                part_v[pl.ds(_L * j, _L)] = v
            pltpu.sync_copy(part_v, out_hbm.at[16])

        @pl.when(sid > _NHT)
        def _():
            splats = gather2(itab_hbm)
            del splats
            loc = jnp.where(sid == _NHT + 1, 8, 9)
            locsplat = plsc.load_gather(
                rbuf, [jnp.full((_L,), _L, jnp.int32) + loc])
            rowv = jnp.full((_L,), 0, jnp.int32) + loc
            for j in range(_D // _L):
                v = plsc.load_gather(
                    rows_v, [rowv, locsplat * _D + (lane + _L * j)])
                part_v[pl.ds(_L * j, _L)] = v

            @pl.when(sid == _NHT + 1)
            def _():
                pltpu.sync_copy(part_v, out_hbm.at[17])

            @pl.when(sid == _NHT + 2)
            def _():
                pltpu.sync_copy(part_v, out_hbm.at[18])


@functools.partial(
    pl.kernel,
    out_type=jax.ShapeDtypeStruct((_L,), jnp.float32),
    mesh=_mesh,
    scratch_types=[
        pltpu.VMEM((24, _D), jnp.float32),
        pltpu.VMEM((_L,), jnp.float32),
        pltpu.VMEM((_L,), jnp.float32),
    ],
    compiler_params=pltpu.CompilerParams(needs_layout_passes=False),
)
def _sc_stage_b(stage_hbm, out_hbm, all_v, out_v, red_v):
    cid = lax.axis_index("c")
    sid = lax.axis_index("s")
    lane = lax.iota(jnp.int32, _L)

    @pl.when(jnp.logical_and(cid == 0, sid == 0))
    def _():
        pltpu.sync_copy(stage_hbm, all_v)
        accp = jnp.zeros((_L,), jnp.float32)
        accn = jnp.zeros((_L,), jnp.float32)
        for j in range(_D // _L):
            sl = slice(_L * j, _L * (j + 1))
            h = all_v[0, sl]
            for t in range(1, _NHT):
                h = h + all_v[t, sl]
            s = all_v[16, sl] + h / jnp.float32(_HIST)
            accp = accp + s * all_v[17, sl]
            accn = accn + s * all_v[18, sl]

        def lane_sum(v):
            for sh in (8, 4, 2, 1):
                red_v[...] = v
                v = v + plsc.load_gather(red_v, [lane ^ sh])
            return v

        ps = lane_sum(accp)
        ns = lane_sum(accn)
        out_v[...] = jnp.where(lane == 0, ps,
                               jnp.where(lane == 1, ns, jnp.float32(0.0)))
        pltpu.sync_copy(out_v, out_hbm)


def kernel(user_id, pos_item_id, neg_item_id, item_history, user_table,
           item_table):
    ids = jnp.concatenate([
        item_history.astype(jnp.int32),
        jnp.asarray(pos_item_id, jnp.int32)[None],
        jnp.asarray(neg_item_id, jnp.int32)[None],
        jnp.full((2,), jnp.asarray(user_id, jnp.int32)),
        jnp.zeros((52,), jnp.int32),
    ])
    blk = ids >> 1
    sub = ids & 1
    ublk = jnp.full((256,), jnp.asarray(user_id, jnp.int32) >> 1)
    padf = (jnp.arange(256, dtype=jnp.int32) >= 200).astype(jnp.int32)
    pidx = jnp.concatenate([blk, sub, ublk, padf])
    utab = user_table.reshape(-1, 2 * _D)
    itab = item_table.reshape(-1, 2 * _D)
    staged = _sc_stage_a(pidx, utab, itab)
    out = _sc_stage_b(staged)
    return (out[0], out[1])


# speedup vs baseline: 15.1032x; 15.1032x over previous
import functools

import jax
import jax.numpy as jnp
from jax import lax
from jax.experimental import pallas as pl
from jax.experimental.pallas import tpu as pltpu
from jax.experimental.pallas import tpu_sc as plsc

_D = 64
_L = 16
_HIST = 200
_RPT = 16
_NHT = 13
_W = 128

_mesh = plsc.VectorSubcoreMesh(core_axis_name="c", subcore_axis_name="s")


@functools.partial(
    pl.kernel,
    out_type=jax.ShapeDtypeStruct((24, _D), jnp.float32),
    mesh=_mesh,
    scratch_types=[
        pltpu.VMEM((_L,), jnp.int32),
        pltpu.VMEM((2 * _L,), jnp.int32),
        pltpu.VMEM((2 * _L,), jnp.int32),
        pltpu.VMEM((_D, _W), jnp.float32),
        pltpu.VMEM((_D, _W), jnp.float32),
        pltpu.VMEM((_D,), jnp.float32),
        pltpu.SemaphoreType.DMA,
    ],
    compiler_params=pltpu.CompilerParams(needs_layout_passes=False),
)
def _sc_stage_a(pidx_hbm, utabT_hbm, itabT_hbm, out_hbm,
                win_v, cbuf, vbuf, blk_v, blk2_v, part_v, sem):
    cid = lax.axis_index("c")
    sid = lax.axis_index("s")
    on0 = cid == 0
    lane = lax.iota(jnp.int32, _L)
    bufs = (blk_v, blk2_v)

    def extract(buf, csplat):
        return [plsc.load_gather(buf, [lane + _L * j, csplat])
                for j in range(_D // _L)]

    @pl.when(jnp.logical_and(on0, sid < _NHT))
    def _():
        base = sid * _RPT
        pltpu.sync_copy(pidx_hbm.at[pl.ds(base, _L)], win_v)
        pltpu.sync_copy(pidx_hbm.at[pl.ds(256 + base, _L)],
                        cbuf.at[pl.ds(_L, _L)])
        pltpu.sync_copy(pidx_hbm.at[pl.ds(512 + base, _L)],
                        vbuf.at[pl.ds(_L, _L)])
        wvec = win_v[...]
        cp = pltpu.async_copy(
            itabT_hbm.at[:, pl.ds(pl.multiple_of(wvec[0], _W), _W)],
            bufs[0], sem)
        acc = [jnp.zeros((_L,), jnp.float32) for _ in range(_D // _L)]
        for i in range(_RPT):
            cp.wait()
            if i + 1 < _RPT:
                cp = pltpu.async_copy(
                    itabT_hbm.at[:, pl.ds(pl.multiple_of(wvec[i + 1], _W), _W)],
                    bufs[(i + 1) % 2], sem)
            csplat = plsc.load_gather(cbuf, [jnp.full((_L,), _L + i, jnp.int32)])
            padspl = plsc.load_gather(vbuf, [jnp.full((_L,), _L + i, jnp.int32)])
            zero = jnp.zeros((_L,), jnp.float32)
            for j, v in enumerate(extract(bufs[i % 2], csplat)):
                acc[j] = acc[j] + jnp.where(padspl == 0, v, zero)
        for j in range(_D // _L):
            part_v[pl.ds(_L * j, _L)] = acc[j]
        pltpu.sync_copy(part_v, out_hbm.at[sid])

    @pl.when(jnp.logical_and(on0, sid >= _NHT))
    def _():
        pltpu.sync_copy(pidx_hbm.at[pl.ds(192, _L)], win_v)
        pltpu.sync_copy(pidx_hbm.at[pl.ds(256 + 192, _L)],
                        cbuf.at[pl.ds(_L, _L)])
        wvec = win_v[...]
        loc = jnp.where(sid == _NHT, 10, jnp.where(sid == _NHT + 1, 8, 9))
        locv = jnp.full((_L,), 0, jnp.int32) + loc
        wbuf = cbuf
        wbuf[pl.ds(0, _L)] = wvec
        wsel = plsc.load_gather(wbuf, [locv])[0]
        csplat = plsc.load_gather(cbuf, [jnp.full((_L,), _L, jnp.int32) + loc])

        @pl.when(sid == _NHT)
        def _():
            pltpu.sync_copy(
                utabT_hbm.at[:, pl.ds(pl.multiple_of(wsel, _W), _W)], blk_v)

        @pl.when(sid > _NHT)
        def _():
            pltpu.sync_copy(
                itabT_hbm.at[:, pl.ds(pl.multiple_of(wsel, _W), _W)], blk_v)

        for j, v in enumerate(extract(blk_v, csplat)):
            part_v[pl.ds(_L * j, _L)] = v

        @pl.when(sid == _NHT)
        def _():
            pltpu.sync_copy(part_v, out_hbm.at[16])

        @pl.when(sid == _NHT + 1)
        def _():
            pltpu.sync_copy(part_v, out_hbm.at[17])

        @pl.when(sid == _NHT + 2)
        def _():
            pltpu.sync_copy(part_v, out_hbm.at[18])


@functools.partial(
    pl.kernel,
    out_type=jax.ShapeDtypeStruct((_L,), jnp.float32),
    mesh=_mesh,
    scratch_types=[
        pltpu.VMEM((24, _D), jnp.float32),
        pltpu.VMEM((_L,), jnp.float32),
        pltpu.VMEM((_L,), jnp.float32),
    ],
    compiler_params=pltpu.CompilerParams(needs_layout_passes=False),
)
def _sc_stage_b(stage_hbm, out_hbm, all_v, out_v, red_v):
    cid = lax.axis_index("c")
    sid = lax.axis_index("s")
    lane = lax.iota(jnp.int32, _L)

    @pl.when(jnp.logical_and(cid == 0, sid == 0))
    def _():
        pltpu.sync_copy(stage_hbm, all_v)
        accp = jnp.zeros((_L,), jnp.float32)
        accn = jnp.zeros((_L,), jnp.float32)
        for j in range(_D // _L):
            sl = slice(_L * j, _L * (j + 1))
            h = all_v[0, sl]
            for t in range(1, _NHT):
                h = h + all_v[t, sl]
            s = all_v[16, sl] + h / jnp.float32(_HIST)
            accp = accp + s * all_v[17, sl]
            accn = accn + s * all_v[18, sl]

        def lane_sum(v):
            for sh in (8, 4, 2, 1):
                red_v[...] = v
                v = v + plsc.load_gather(red_v, [lane ^ sh])
            return v

        ps = lane_sum(accp)
        ns = lane_sum(accn)
        out_v[...] = jnp.where(lane == 0, ps,
                               jnp.where(lane == 1, ns, jnp.float32(0.0)))
        pltpu.sync_copy(out_v, out_hbm)


def kernel(user_id, pos_item_id, neg_item_id, item_history, user_table,
           item_table):
    ids = jnp.concatenate([
        item_history.astype(jnp.int32),
        jnp.asarray(pos_item_id, jnp.int32)[None],
        jnp.asarray(neg_item_id, jnp.int32)[None],
        jnp.full((2,), jnp.asarray(user_id, jnp.int32)),
        jnp.zeros((52,), jnp.int32),
    ])
    win = (ids >> 7) << 7
    col = ids & (_W - 1)
    padf = (jnp.arange(256, dtype=jnp.int32) >= 200).astype(jnp.int32)
    pidx = jnp.concatenate([win, col, padf, jnp.zeros((256,), jnp.int32)])
    utabT = user_table.T
    itabT = item_table.T
    staged = _sc_stage_a(pidx, utabT, itabT)
    out = _sc_stage_b(staged)
    return (out[0], out[1])


# speedup vs baseline: 21.3189x; 1.4115x over previous
import functools

import jax
import jax.numpy as jnp
from jax import lax
from jax.experimental import pallas as pl
from jax.experimental.pallas import tpu as pltpu
from jax.experimental.pallas import tpu_sc as plsc

_D = 64
_L = 16
_HIST = 200
_RPT = 8
_NHT = 13
_NBUF = 4
_W = 128

_mesh = plsc.VectorSubcoreMesh(core_axis_name="c", subcore_axis_name="s")


@functools.partial(
    pl.kernel,
    out_type=jax.ShapeDtypeStruct((32, _D), jnp.float32),
    mesh=_mesh,
    scratch_types=[
        pltpu.VMEM((_L,), jnp.int32),
        pltpu.VMEM((2 * _L,), jnp.int32),
        pltpu.VMEM((2 * _L,), jnp.int32),
        pltpu.VMEM((_NBUF, _D, _W), jnp.float32),
        pltpu.VMEM((_D,), jnp.float32),
        pltpu.SemaphoreType.DMA,
    ],
    compiler_params=pltpu.CompilerParams(needs_layout_passes=False),
)
def _sc_stage_a(pidx_hbm, utabT_hbm, itabT_hbm, out_hbm,
                win_v, cbuf, vbuf, blk_r, part_v, sem):
    cid = lax.axis_index("c")
    sid = lax.axis_index("s")
    on0 = cid == 0
    lane = lax.iota(jnp.int32, _L)
    bufs = [blk_r.at[b] for b in range(_NBUF)]

    def extract(buf, csplat):
        return [plsc.load_gather(buf, [lane + _L * j, csplat])
                for j in range(_D // _L)]

    wid = cid * _NHT + sid
    @pl.when(sid < _NHT)
    def _():
        base = wid * _RPT
        pltpu.sync_copy(pidx_hbm.at[pl.ds(base, _L)], win_v)
        pltpu.sync_copy(pidx_hbm.at[pl.ds(256 + base, _L)],
                        cbuf.at[pl.ds(_L, _L)])
        pltpu.sync_copy(pidx_hbm.at[pl.ds(512 + base, _L)],
                        vbuf.at[pl.ds(_L, _L)])
        wvec = win_v[...]
        cps = []
        for b in range(_NBUF - 1):
            cps.append(pltpu.async_copy(
                itabT_hbm.at[:, pl.ds(pl.multiple_of(wvec[b], _W), _W)],
                bufs[b], sem))
        acc = [jnp.zeros((_L,), jnp.float32) for _ in range(_D // _L)]
        for i in range(_RPT):
            nxt = i + _NBUF - 1
            if nxt < _RPT:
                cps.append(pltpu.async_copy(
                    itabT_hbm.at[:, pl.ds(pl.multiple_of(wvec[nxt], _W), _W)],
                    bufs[nxt % _NBUF], sem))
            cps[i].wait()
            csplat = plsc.load_gather(cbuf, [jnp.full((_L,), _L + i, jnp.int32)])
            padspl = plsc.load_gather(vbuf, [jnp.full((_L,), _L + i, jnp.int32)])
            zero = jnp.zeros((_L,), jnp.float32)
            for j, v in enumerate(extract(bufs[i % _NBUF], csplat)):
                acc[j] = acc[j] + jnp.where(padspl == 0, v, zero)
        for j in range(_D // _L):
            part_v[pl.ds(_L * j, _L)] = acc[j]
        pltpu.sync_copy(part_v, out_hbm.at[wid])

    @pl.when(jnp.logical_and(on0, sid >= _NHT))
    def _():
        pltpu.sync_copy(pidx_hbm.at[pl.ds(192, _L)], win_v)
        pltpu.sync_copy(pidx_hbm.at[pl.ds(256 + 192, _L)],
                        cbuf.at[pl.ds(_L, _L)])
        wvec = win_v[...]
        loc = jnp.where(sid == _NHT, 10, jnp.where(sid == _NHT + 1, 8, 9))
        locv = jnp.full((_L,), 0, jnp.int32) + loc
        wbuf = cbuf
        wbuf[pl.ds(0, _L)] = wvec
        wsel = plsc.load_gather(wbuf, [locv])[0]
        csplat = plsc.load_gather(cbuf, [jnp.full((_L,), _L, jnp.int32) + loc])

        @pl.when(sid == _NHT)
        def _():
            pltpu.sync_copy(
                utabT_hbm.at[:, pl.ds(pl.multiple_of(wsel, _W), _W)], bufs[0])

        @pl.when(sid > _NHT)
        def _():
            pltpu.sync_copy(
                itabT_hbm.at[:, pl.ds(pl.multiple_of(wsel, _W), _W)], bufs[0])

        for j, v in enumerate(extract(bufs[0], csplat)):
            part_v[pl.ds(_L * j, _L)] = v

        @pl.when(sid == _NHT)
        def _():
            pltpu.sync_copy(part_v, out_hbm.at[26])

        @pl.when(sid == _NHT + 1)
        def _():
            pltpu.sync_copy(part_v, out_hbm.at[27])

        @pl.when(sid == _NHT + 2)
        def _():
            pltpu.sync_copy(part_v, out_hbm.at[28])


@functools.partial(
    pl.kernel,
    out_type=jax.ShapeDtypeStruct((_L,), jnp.float32),
    mesh=_mesh,
    scratch_types=[
        pltpu.VMEM((32, _D), jnp.float32),
        pltpu.VMEM((_L,), jnp.float32),
        pltpu.VMEM((_L,), jnp.float32),
    ],
    compiler_params=pltpu.CompilerParams(needs_layout_passes=False),
)
def _sc_stage_b(stage_hbm, out_hbm, all_v, out_v, red_v):
    cid = lax.axis_index("c")
    sid = lax.axis_index("s")
    lane = lax.iota(jnp.int32, _L)

    @pl.when(jnp.logical_and(cid == 0, sid == 0))
    def _():
        pltpu.sync_copy(stage_hbm, all_v)
        accp = jnp.zeros((_L,), jnp.float32)
        accn = jnp.zeros((_L,), jnp.float32)
        for j in range(_D // _L):
            sl = slice(_L * j, _L * (j + 1))
            h = all_v[0, sl]
            for t in range(1, 2 * _NHT):
                h = h + all_v[t, sl]
            s = all_v[26, sl] + h / jnp.float32(_HIST)
            accp = accp + s * all_v[27, sl]
            accn = accn + s * all_v[28, sl]

        def lane_sum(v):
            for sh in (8, 4, 2, 1):
                red_v[...] = v
                v = v + plsc.load_gather(red_v, [lane ^ sh])
            return v

        ps = lane_sum(accp)
        ns = lane_sum(accn)
        out_v[...] = jnp.where(lane == 0, ps,
                               jnp.where(lane == 1, ns, jnp.float32(0.0)))
        pltpu.sync_copy(out_v, out_hbm)


def kernel(user_id, pos_item_id, neg_item_id, item_history, user_table,
           item_table):
    ids = jnp.concatenate([
        item_history.astype(jnp.int32),
        jnp.asarray(pos_item_id, jnp.int32)[None],
        jnp.asarray(neg_item_id, jnp.int32)[None],
        jnp.full((2,), jnp.asarray(user_id, jnp.int32)),
        jnp.zeros((52,), jnp.int32),
    ])
    win = (ids >> 7) << 7
    col = ids & (_W - 1)
    padf = (jnp.arange(256, dtype=jnp.int32) >= 200).astype(jnp.int32)
    pidx = jnp.concatenate([win, col, padf, jnp.zeros((256,), jnp.int32)])
    utabT = user_table.T
    itabT = item_table.T
    staged = _sc_stage_a(pidx, utabT, itabT)
    out = _sc_stage_b(staged)
    return (out[0], out[1])


# speedup vs baseline: 23.4410x; 1.0995x over previous
import functools

import jax
import jax.numpy as jnp
from jax import lax
from jax.experimental import pallas as pl
from jax.experimental.pallas import tpu as pltpu
from jax.experimental.pallas import tpu_sc as plsc

_D = 64
_L = 16
_HIST = 200
_RPT = 8
_NHT = 13
_NBUF = 4
_W = 128

_mesh = plsc.VectorSubcoreMesh(core_axis_name="c", subcore_axis_name="s")


@functools.partial(
    pl.kernel,
    out_type=jax.ShapeDtypeStruct((32, _D), jnp.float32),
    mesh=_mesh,
    scratch_types=[
        pltpu.VMEM((_L,), jnp.int32),
        pltpu.VMEM((2 * _L,), jnp.int32),
        pltpu.VMEM((2 * _L,), jnp.int32),
        pltpu.VMEM((_NBUF, _D, _W), jnp.float32),
        pltpu.VMEM((_D,), jnp.float32),
        pltpu.SemaphoreType.DMA,
    ],
    compiler_params=pltpu.CompilerParams(needs_layout_passes=False),
)
def _sc_stage_a(pidx_hbm, utabT_hbm, itabT_hbm, out_hbm,
                win_v, cbuf, vbuf, blk_r, part_v, sem):
    cid = lax.axis_index("c")
    sid = lax.axis_index("s")
    on0 = cid == 0
    lane = lax.iota(jnp.int32, _L)
    bufs = [blk_r.at[b] for b in range(_NBUF)]

    def extract(buf, csplat):
        return [plsc.load_gather(buf, [lane + _L * j, csplat])
                for j in range(_D // _L)]

    wid = cid * _NHT + sid
    @pl.when(sid < _NHT)
    def _():
        base = wid * _RPT
        pltpu.sync_copy(pidx_hbm.at[pl.ds(base, _L)], win_v)
        pltpu.sync_copy(pidx_hbm.at[pl.ds(256 + base, _L)],
                        cbuf.at[pl.ds(_L, _L)])
        pltpu.sync_copy(pidx_hbm.at[pl.ds(512 + base, _L)],
                        vbuf.at[pl.ds(_L, _L)])
        wvec = win_v[...]
        cps = []
        for b in range(_NBUF - 1):
            cps.append(pltpu.async_copy(
                itabT_hbm.at[:, pl.ds(pl.multiple_of(wvec[b], _W), _W)],
                bufs[b], sem))
        acc = [jnp.zeros((_L,), jnp.float32) for _ in range(_D // _L)]
        for i in range(_RPT):
            nxt = i + _NBUF - 1
            if nxt < _RPT:
                cps.append(pltpu.async_copy(
                    itabT_hbm.at[:, pl.ds(pl.multiple_of(wvec[nxt], _W), _W)],
                    bufs[nxt % _NBUF], sem))
            cps[i].wait()
            csplat = plsc.load_gather(cbuf, [jnp.full((_L,), _L + i, jnp.int32)])
            padspl = plsc.load_gather(vbuf, [jnp.full((_L,), _L + i, jnp.int32)])
            zero = jnp.zeros((_L,), jnp.float32)
            for j, v in enumerate(extract(bufs[i % _NBUF], csplat)):
                acc[j] = acc[j] + jnp.where(padspl == 0, v, zero)
        for j in range(_D // _L):
            part_v[pl.ds(_L * j, _L)] = acc[j]
        pltpu.sync_copy(part_v, out_hbm.at[wid])

    @pl.when(jnp.logical_and(on0, sid >= _NHT))
    def _():
        pltpu.sync_copy(pidx_hbm.at[pl.ds(192, _L)], win_v)
        pltpu.sync_copy(pidx_hbm.at[pl.ds(256 + 192, _L)],
                        cbuf.at[pl.ds(_L, _L)])
        wvec = win_v[...]
        loc = jnp.where(sid == _NHT, 10, jnp.where(sid == _NHT + 1, 8, 9))
        locv = jnp.full((_L,), 0, jnp.int32) + loc
        wbuf = cbuf
        wbuf[pl.ds(0, _L)] = wvec
        wsel = plsc.load_gather(wbuf, [locv])[0]
        csplat = plsc.load_gather(cbuf, [jnp.full((_L,), _L, jnp.int32) + loc])

        @pl.when(sid == _NHT)
        def _():
            pltpu.sync_copy(
                utabT_hbm.at[:, pl.ds(pl.multiple_of(wsel, _W), _W)], bufs[0])

        @pl.when(sid > _NHT)
        def _():
            pltpu.sync_copy(
                itabT_hbm.at[:, pl.ds(pl.multiple_of(wsel, _W), _W)], bufs[0])

        for j, v in enumerate(extract(bufs[0], csplat)):
            part_v[pl.ds(_L * j, _L)] = v

        @pl.when(sid == _NHT)
        def _():
            pltpu.sync_copy(part_v, out_hbm.at[26])

        @pl.when(sid == _NHT + 1)
        def _():
            pltpu.sync_copy(part_v, out_hbm.at[27])

        @pl.when(sid == _NHT + 2)
        def _():
            pltpu.sync_copy(part_v, out_hbm.at[28])


def _tc_stage_b_body(x_ref, o_ref):
    x = x_ref[...]
    h = jnp.sum(x[0:2 * _NHT, :], axis=0)
    s = x[26, :] + h / jnp.float32(_HIST)
    ps = jnp.sum(s * x[27, :])
    ns = jnp.sum(s * x[28, :])
    r = jax.lax.broadcasted_iota(jnp.int32, (8, 128), 0)
    c = jax.lax.broadcasted_iota(jnp.int32, (8, 128), 1)
    o_ref[...] = jnp.where((r == 0) & (c == 0), ps,
                           jnp.where((r == 0) & (c == 1), ns,
                                     jnp.float32(0.0)))


_tc_stage_b = pl.pallas_call(
    _tc_stage_b_body,
    out_shape=jax.ShapeDtypeStruct((8, 128), jnp.float32),
)


def kernel(user_id, pos_item_id, neg_item_id, item_history, user_table,
           item_table):
    ids = jnp.concatenate([
        item_history.astype(jnp.int32),
        jnp.asarray(pos_item_id, jnp.int32)[None],
        jnp.asarray(neg_item_id, jnp.int32)[None],
        jnp.full((2,), jnp.asarray(user_id, jnp.int32)),
        jnp.zeros((52,), jnp.int32),
    ])
    win = (ids >> 7) << 7
    col = ids & (_W - 1)
    padf = (jnp.arange(256, dtype=jnp.int32) >= 200).astype(jnp.int32)
    pidx = jnp.concatenate([win, col, padf, jnp.zeros((256,), jnp.int32)])
    utabT = user_table.T
    itabT = item_table.T
    staged = _sc_stage_a(pidx, utabT, itabT)
    out = _tc_stage_b(staged)
    return (out[0, 0], out[0, 1])


# speedup vs baseline: 23.5649x; 1.0053x over previous
import functools

import jax
import jax.numpy as jnp
from jax import lax
from jax.experimental import pallas as pl
from jax.experimental.pallas import tpu as pltpu
from jax.experimental.pallas import tpu_sc as plsc

_D = 64
_L = 16
_HIST = 200
_RPT = 8
_NHT = 13
_NBUF = 4
_W = 128

_mesh = plsc.VectorSubcoreMesh(core_axis_name="c", subcore_axis_name="s")


@functools.partial(
    pl.kernel,
    out_type=jax.ShapeDtypeStruct((32, _D), jnp.float32),
    mesh=_mesh,
    scratch_types=[
        pltpu.VMEM((_L,), jnp.int32),
        pltpu.VMEM((2 * _L,), jnp.int32),
        pltpu.VMEM((2 * _L,), jnp.int32),
        pltpu.VMEM((_NBUF, _D, _W), jnp.float32),
        pltpu.VMEM((_D,), jnp.float32),
        pltpu.SemaphoreType.DMA,
    ],
    compiler_params=pltpu.CompilerParams(needs_layout_passes=False),
)
def _sc_stage_a(pidx_hbm, utabT_hbm, itabT_hbm, out_hbm,
                win_v, cbuf, vbuf, blk_r, part_v, sem):
    cid = lax.axis_index("c")
    sid = lax.axis_index("s")
    on0 = cid == 0
    lane = lax.iota(jnp.int32, _L)
    bufs = [blk_r.at[b] for b in range(_NBUF)]

    def extract(buf, csplat):
        return [plsc.load_gather(buf, [lane + _L * j, csplat])
                for j in range(_D // _L)]

    wid = cid * _NHT + sid
    @pl.when(sid < _NHT)
    def _():
        base = wid * _RPT
        pltpu.sync_copy(pidx_hbm.at[pl.ds(base, _L)], win_v)
        pltpu.sync_copy(pidx_hbm.at[pl.ds(256 + base, _L)],
                        cbuf.at[pl.ds(_L, _L)])
        pltpu.sync_copy(pidx_hbm.at[pl.ds(512 + base, _L)],
                        vbuf.at[pl.ds(_L, _L)])
        wvec = win_v[...]
        cps = []
        for b in range(_NBUF - 1):
            cps.append(pltpu.async_copy(
                itabT_hbm.at[:, pl.ds(pl.multiple_of(wvec[b], _W), _W)],
                bufs[b], sem))
        acc = [jnp.zeros((_L,), jnp.float32) for _ in range(_D // _L)]
        for i in range(_RPT):
            nxt = i + _NBUF - 1
            if nxt < _RPT:
                cps.append(pltpu.async_copy(
                    itabT_hbm.at[:, pl.ds(pl.multiple_of(wvec[nxt], _W), _W)],
                    bufs[nxt % _NBUF], sem))
            cps[i].wait()
            csplat = plsc.load_gather(cbuf, [jnp.full((_L,), _L + i, jnp.int32)])
            padspl = plsc.load_gather(vbuf, [jnp.full((_L,), _L + i, jnp.int32)])
            zero = jnp.zeros((_L,), jnp.float32)
            for j, v in enumerate(extract(bufs[i % _NBUF], csplat)):
                acc[j] = acc[j] + jnp.where(padspl == 0, v, zero)
        for j in range(_D // _L):
            part_v[pl.ds(_L * j, _L)] = acc[j]
        pltpu.sync_copy(part_v, out_hbm.at[wid])

    @pl.when(jnp.logical_and(on0, sid >= _NHT))
    def _():
        pltpu.sync_copy(pidx_hbm.at[pl.ds(192, _L)], win_v)
        pltpu.sync_copy(pidx_hbm.at[pl.ds(256 + 192, _L)],
                        cbuf.at[pl.ds(_L, _L)])
        wvec = win_v[...]
        loc = jnp.where(sid == _NHT, 10, jnp.where(sid == _NHT + 1, 8, 9))
        locv = jnp.full((_L,), 0, jnp.int32) + loc
        wbuf = cbuf
        wbuf[pl.ds(0, _L)] = wvec
        wsel = plsc.load_gather(wbuf, [locv])[0]
        csplat = plsc.load_gather(cbuf, [jnp.full((_L,), _L, jnp.int32) + loc])

        @pl.when(sid == _NHT)
        def _():
            pltpu.sync_copy(
                utabT_hbm.at[:, pl.ds(pl.multiple_of(wsel, _W), _W)], bufs[0])

        @pl.when(sid > _NHT)
        def _():
            pltpu.sync_copy(
                itabT_hbm.at[:, pl.ds(pl.multiple_of(wsel, _W), _W)], bufs[0])

        for j, v in enumerate(extract(bufs[0], csplat)):
            part_v[pl.ds(_L * j, _L)] = v

        @pl.when(sid == _NHT)
        def _():
            pltpu.sync_copy(part_v, out_hbm.at[26])

        @pl.when(sid == _NHT + 1)
        def _():
            pltpu.sync_copy(part_v, out_hbm.at[27])

        @pl.when(sid == _NHT + 2)
        def _():
            pltpu.sync_copy(part_v, out_hbm.at[28])


def _tc_stage_b_body(x_ref, o_ref):
    x = x_ref[...]
    h = jnp.sum(x[0:2 * _NHT, :], axis=0)
    s = x[26, :] + h / jnp.float32(_HIST)
    ps = jnp.sum(s * x[27, :])
    ns = jnp.sum(s * x[28, :])
    r = jax.lax.broadcasted_iota(jnp.int32, (8, 128), 0)
    c = jax.lax.broadcasted_iota(jnp.int32, (8, 128), 1)
    o_ref[...] = jnp.where((r == 0) & (c == 0), ps,
                           jnp.where((r == 0) & (c == 1), ns,
                                     jnp.float32(0.0)))


_tc_stage_b = pl.pallas_call(
    _tc_stage_b_body,
    out_shape=jax.ShapeDtypeStruct((8, 128), jnp.float32),
)


def kernel(user_id, pos_item_id, neg_item_id, item_history, user_table,
           item_table):
    ids = jnp.concatenate([
        item_history.astype(jnp.int32),
        jnp.asarray(pos_item_id, jnp.int32)[None],
        jnp.asarray(neg_item_id, jnp.int32)[None],
        jnp.full((2,), jnp.asarray(user_id, jnp.int32)),
        jnp.zeros((52,), jnp.int32),
    ])
    win = (ids >> 7) << 7
    col = ids & (_W - 1)
    padf = (jnp.arange(256, dtype=jnp.int32) >= 200).astype(jnp.int32)
    pidx = jnp.concatenate([win, col, padf])
    utabT = user_table.T
    itabT = item_table.T
    staged = _sc_stage_a(pidx, utabT, itabT)
    out = _tc_stage_b(staged)
    return (out[0, 0], out[0, 1])
